# baseline (speedup 1.0000x reference)
import functools

import jax
import jax.numpy as jnp
from jax import lax
from jax.experimental import pallas as pl
from jax.experimental.pallas import tpu as pltpu
from jax.experimental.pallas import tpu_sc as plsc

N = 10000
NPAD = 10240
E = 320000
CHUNK = 128
NW = 32
CPW = 80
EPAD = NW * CPW * CHUNK
DUMMY = N
STRIPE = NPAD // 16

_MESH = plsc.VectorSubcoreMesh(core_axis_name="c", subcore_axis_name="s")


def _fill_rows(ref, n_rows, n_cols, value):
    vec = jnp.full((16,), value, jnp.float32)

    def body(i, _):
        for l in range(n_cols // 16):
            ref[i, pl.ds(16 * l, 16)] = vec
        return 0

    lax.fori_loop(0, n_rows, body, 0)


def _make_deg_kernel():
    @functools.partial(
        pl.kernel,
        out_type=jax.ShapeDtypeStruct((2, NPAD, 16), jnp.float32),
        mesh=_MESH,
        compiler_params=pltpu.CompilerParams(use_tc_tiling_on_sc=False),
        scratch_types=[
            pltpu.VMEM((CPW, CHUNK), jnp.int32),
            pltpu.VMEM((CHUNK, 16), jnp.float32),
            pltpu.VMEM((CHUNK, 16), jnp.float32),
            pltpu.VMEM_SHARED((NPAD, 16), jnp.float32),
            pltpu.SemaphoreType.DMA,
        ],
    )
    def deg_kernel(dst_hbm, out_hbm, dst_v, ones_v, zero_v, acc_sh, dsem):
        c = lax.axis_index("c")
        s = lax.axis_index("s")
        wid = c * 16 + s

        _fill_rows(zero_v, CHUNK, 16, 0.0)
        _fill_rows(ones_v, CHUNK, 16, 1.0)
        for t in range(STRIPE // CHUNK):
            pltpu.sync_copy(zero_v, acc_sh.at[pl.ds(s * STRIPE + t * CHUNK, CHUNK)])
        plsc.subcore_barrier()

        pltpu.sync_copy(dst_hbm.at[pl.ds(wid * CPW, CPW)], dst_v)

        def body(j, _):
            pltpu.async_copy(ones_v, acc_sh.at[dst_v.at[j]], dsem, add=True)
            return 0

        lax.fori_loop(0, CPW, body, 0)

        def drain(j, _):
            pltpu.make_async_copy(ones_v, acc_sh.at[dst_v.at[j]], dsem).wait()
            return 0

        lax.fori_loop(0, CPW, drain, 0)
        plsc.subcore_barrier()
        pltpu.sync_copy(acc_sh.at[pl.ds(s * STRIPE, STRIPE)],
                        out_hbm.at[c, pl.ds(s * STRIPE, STRIPE)])

    return deg_kernel


NBUF = 4
CPW0 = 80
CPW1 = 80
CPWMAX = max(CPW0, CPW1)


def _make_edge_scatter(D, bf16_gather):
    gdt = jnp.int32 if bf16_gather else jnp.float32
    @functools.partial(
        pl.kernel,
        out_type=jax.ShapeDtypeStruct((2, NPAD, D), jnp.float32),
        mesh=_MESH,
        compiler_params=pltpu.CompilerParams(use_tc_tiling_on_sc=False),
        scratch_types=[
            pltpu.VMEM((CPWMAX, CHUNK), jnp.int32),
            pltpu.VMEM((CPWMAX, CHUNK), jnp.int32),
            pltpu.VMEM((NBUF, CHUNK, D), jnp.float32),
            pltpu.VMEM((NBUF, CHUNK, D // 2) if bf16_gather else (1, 16, D), gdt),
            pltpu.VMEM_SHARED((NPAD, D), jnp.float32),
            pltpu.VMEM_SHARED((NPAD, D // 2 if bf16_gather else D), gdt),
        ] + [pltpu.SemaphoreType.DMA] * NBUF,
    )
    def edge_kernel(y_hbm, src_hbm, dst_hbm, out_hbm, src_v, dst_v, rows_v, gbuf, acc_sh, ytab_sh, *sems):
        c = lax.axis_index("c")
        s = lax.axis_index("s")

        vec = jnp.zeros((16,), jnp.float32)

        def zbody(i, _):
            for l in range(D // 16):
                rows_v[0, i, pl.ds(16 * l, 16)] = vec
            return 0

        lax.fori_loop(0, CHUNK, zbody, 0)
        for t in range(STRIPE // CHUNK):
            pltpu.sync_copy(rows_v.at[0], acc_sh.at[pl.ds(s * STRIPE + t * CHUNK, CHUNK)])
        pltpu.sync_copy(y_hbm.at[pl.ds(s * STRIPE, STRIPE)],
                        ytab_sh.at[pl.ds(s * STRIPE, STRIPE)])
        plsc.subcore_barrier()

        gt = gbuf if bf16_gather else rows_v

        def g_src(p):
            return ytab_sh if (p == 0 or not bf16_gather) else y_hbm

        def g_start(j, p):
            pltpu.async_copy(g_src(p).at[src_v.at[j]], gt.at[p], sems[p])

        def g_wait(j, p):
            pltpu.make_async_copy(g_src(p).at[src_v.at[j]], gt.at[p], sems[p]).wait()

        def s_start(j, p):
            pltpu.async_copy(rows_v.at[p], acc_sh.at[dst_v.at[j]], sems[p], add=True)

        def s_wait(j, p):
            pltpu.make_async_copy(rows_v.at[p], acc_sh.at[dst_v.at[j]], sems[p]).wait()

        mask_hi = jnp.full((16,), -65536, jnp.int32)

        def convert(p):
            def cbody(i, _):
                for g in range(D // 32):
                    w = gbuf[p, i, pl.ds(16 * g, 16)]
                    lo = lax.bitcast_convert_type(w << 16, jnp.float32)
                    hi = lax.bitcast_convert_type(w & mask_hi, jnp.float32)
                    rows_v[p, i, pl.ds(32 * g, 16)] = lo
                    rows_v[p, i, pl.ds(32 * g + 16, 16)] = hi
                return 0

            lax.fori_loop(0, CHUNK, cbody, 0)

        def run(base, n):
            pltpu.sync_copy(src_hbm.at[pl.ds(base, n)], src_v.at[pl.ds(0, n)])
            pltpu.sync_copy(dst_hbm.at[pl.ds(base, n)], dst_v.at[pl.ds(0, n)])

            for p in range(NBUF):
                g_start(p, p)

            def body(g, _):
                j0 = g * NBUF
                for p in range(NBUF):
                    g_wait(j0 + p, p)
                    if bf16_gather:
                        convert(p)
                    s_start(j0 + p, p)
                for p in range(NBUF):
                    s_wait(j0 + p, p)
                    g_start(j0 + NBUF + p, p)
                return 0

            lax.fori_loop(0, n // NBUF - 1, body, 0)

            jlast = n - NBUF
            for p in range(NBUF):
                g_wait(jlast + p, p)
                if bf16_gather:
                    convert(p)
                s_start(jlast + p, p)
            for p in range(NBUF):
                s_wait(jlast + p, p)

        if CPW0 > 0:
            @pl.when(c == 0)
            def _():
                run(s * CPW0, CPW0)

        if CPW1 > 0:
            @pl.when(c == 1)
            def _():
                run(16 * CPW0 + s * CPW1, CPW1)

        plsc.subcore_barrier()
        pltpu.sync_copy(acc_sh.at[pl.ds(s * STRIPE, STRIPE)],
                        out_hbm.at[c, pl.ds(s * STRIPE, STRIPE)])

    return edge_kernel


_deg_scatter = _make_deg_kernel()
_edge_scatter64 = _make_edge_scatter(64, bf16_gather=True)
_edge_scatter16 = _make_edge_scatter(16, bf16_gather=False)


def _tc1_body(degp_ref, x_ref, w1_ref, dis_ref, y1_ref, y1bf_ref):
    deg = degp_ref[0, :, 0:1] + degp_ref[1, :, 0:1] + 1.0
    dis = lax.rsqrt(deg)
    dis_ref[...] = jnp.broadcast_to(dis, (NPAD, 16))
    h = jnp.dot(x_ref[...], w1_ref[...], preferred_element_type=jnp.float32)
    y1 = dis * h
    y1_ref[...] = y1
    b16 = lax.bitcast_convert_type(y1.astype(jnp.bfloat16), jnp.int16)
    w32 = b16.astype(jnp.int32) & 0xFFFF
    words = [w32[:, 32 * g:32 * g + 16] | (w32[:, 32 * g + 16:32 * g + 32] << 16)
             for g in range(2)]
    y1bf_ref[...] = jnp.concatenate(words, axis=1)


def _tc2_body(m1p_ref, y1_ref, dis_ref, b1_ref, w2_ref, y2_ref):
    dis = dis_ref[:, 0:1]
    pre = dis * (m1p_ref[0] + m1p_ref[1] + y1_ref[...]) + b1_ref[...]
    h1 = jnp.maximum(pre, 0.0)
    g = jnp.dot(h1, w2_ref[...], preferred_element_type=jnp.float32)
    y2_ref[...] = dis * g


def _tc3_body(m2p_ref, y2_ref, dis_ref, b2_ref, z_ref):
    dis = dis_ref[:, 0:1]
    z_ref[...] = dis * (m2p_ref[0] + m2p_ref[1] + y2_ref[...]) + b2_ref[...]


@jax.jit
def kernel(x, edge_index, W1, b1, W2, b2):
    f32 = jnp.float32
    xp = jnp.zeros((NPAD, 128), f32).at[:N].set(x)
    w1p = jnp.zeros((128, 64), f32).at[:, :50].set(W1)
    b1p = jnp.zeros((1, 64), f32).at[0, :50].set(b1)
    w2p = jnp.zeros((64, 16), f32).at[:50, :10].set(W2)
    b2p = jnp.zeros((1, 16), f32).at[0, :10].set(b2)

    pad = jnp.full((EPAD - E,), DUMMY, jnp.int32)
    src2d = jnp.concatenate([edge_index[0].astype(jnp.int32), pad]).reshape(EPAD // CHUNK, CHUNK)
    dst2d = jnp.concatenate([edge_index[1].astype(jnp.int32), pad]).reshape(EPAD // CHUNK, CHUNK)

    degp = _deg_scatter(dst2d)

    dis, y1, y1bf = pl.pallas_call(
        _tc1_body,
        out_shape=(jax.ShapeDtypeStruct((NPAD, 16), f32),
                   jax.ShapeDtypeStruct((NPAD, 64), f32),
                   jax.ShapeDtypeStruct((NPAD, 32), jnp.int32)),
    )(degp, xp, w1p)

    m1p = _edge_scatter64(y1bf, src2d, dst2d)

    y2 = pl.pallas_call(
        _tc2_body,
        out_shape=jax.ShapeDtypeStruct((NPAD, 16), f32),
    )(m1p, y1, dis, b1p, w2p)

    m2p = _edge_scatter16(y2, src2d, dst2d)

    z = pl.pallas_call(
        _tc3_body,
        out_shape=jax.ShapeDtypeStruct((NPAD, 16), f32),
    )(m2p, y2, dis, b2p)

    return z[:N, :10]


# speedup vs baseline: 1.0625x; 1.0625x over previous
import functools

import jax
import jax.numpy as jnp
from jax import lax
from jax.experimental import pallas as pl
from jax.experimental.pallas import tpu as pltpu
from jax.experimental.pallas import tpu_sc as plsc

N = 10000
NPAD = 10240
E = 320000
CHUNK = 128
NW = 32
CPW = 80
EPAD = NW * CPW * CHUNK
DUMMY = N
STRIPE = NPAD // 16

_MESH = plsc.VectorSubcoreMesh(core_axis_name="c", subcore_axis_name="s")


def _fill_rows(ref, n_rows, n_cols, value):
    vec = jnp.full((16,), value, jnp.float32)

    def body(i, _):
        for l in range(n_cols // 16):
            ref[i, pl.ds(16 * l, 16)] = vec
        return 0

    lax.fori_loop(0, n_rows, body, 0)


def _make_deg_kernel():
    @functools.partial(
        pl.kernel,
        out_type=jax.ShapeDtypeStruct((2, NPAD, 16), jnp.float32),
        mesh=_MESH,
        compiler_params=pltpu.CompilerParams(use_tc_tiling_on_sc=False),
        scratch_types=[
            pltpu.VMEM((CPW, CHUNK), jnp.int32),
            pltpu.VMEM((CHUNK, 16), jnp.float32),
            pltpu.VMEM((CHUNK, 16), jnp.float32),
            pltpu.VMEM_SHARED((NPAD, 16), jnp.float32),
            pltpu.SemaphoreType.DMA,
        ],
    )
    def deg_kernel(dst_hbm, out_hbm, dst_v, ones_v, zero_v, acc_sh, dsem):
        c = lax.axis_index("c")
        s = lax.axis_index("s")
        wid = c * 16 + s

        _fill_rows(zero_v, CHUNK, 16, 0.0)
        _fill_rows(ones_v, CHUNK, 16, 1.0)
        for t in range(STRIPE // CHUNK):
            pltpu.sync_copy(zero_v, acc_sh.at[pl.ds(s * STRIPE + t * CHUNK, CHUNK)])
        plsc.subcore_barrier()

        pltpu.sync_copy(dst_hbm.at[pl.ds(wid * CPW, CPW)], dst_v)

        def body(j, _):
            pltpu.async_copy(ones_v, acc_sh.at[dst_v.at[j]], dsem, add=True)
            return 0

        lax.fori_loop(0, CPW, body, 0)

        def drain(j, _):
            pltpu.make_async_copy(ones_v, acc_sh.at[dst_v.at[j]], dsem).wait()
            return 0

        lax.fori_loop(0, CPW, drain, 0)
        plsc.subcore_barrier()
        pltpu.sync_copy(acc_sh.at[pl.ds(s * STRIPE, STRIPE)],
                        out_hbm.at[c, pl.ds(s * STRIPE, STRIPE)])

    return deg_kernel


NBUF = 4
CPW0 = 80
CPW1 = 80
CPWMAX = max(CPW0, CPW1)


def _make_edge_scatter(D, bf16_gather):
    gdt = jnp.int32 if bf16_gather else jnp.float32
    @functools.partial(
        pl.kernel,
        out_type=jax.ShapeDtypeStruct((2, NPAD, D), jnp.float32),
        mesh=_MESH,
        compiler_params=pltpu.CompilerParams(use_tc_tiling_on_sc=False),
        scratch_types=[
            pltpu.VMEM((CPWMAX, CHUNK), jnp.int32),
            pltpu.VMEM((CPWMAX, CHUNK), jnp.int32),
            pltpu.VMEM((NBUF, CHUNK, D), jnp.float32),
            pltpu.VMEM((NBUF, CHUNK, D // 2) if bf16_gather else (1, 16, D), gdt),
            pltpu.VMEM_SHARED((NPAD, D), jnp.float32),
            pltpu.VMEM_SHARED((NPAD, D // 2 if bf16_gather else D), gdt),
        ] + [pltpu.SemaphoreType.DMA] * NBUF,
    )
    def edge_kernel(y_hbm, src_hbm, dst_hbm, out_hbm, src_v, dst_v, rows_v, gbuf, acc_sh, ytab_sh, *sems):
        c = lax.axis_index("c")
        s = lax.axis_index("s")

        vec = jnp.zeros((16,), jnp.float32)

        def zbody(i, _):
            for l in range(D // 16):
                rows_v[0, i, pl.ds(16 * l, 16)] = vec
            return 0

        lax.fori_loop(0, CHUNK, zbody, 0)
        for t in range(STRIPE // CHUNK):
            pltpu.sync_copy(rows_v.at[0], acc_sh.at[pl.ds(s * STRIPE + t * CHUNK, CHUNK)])
        pltpu.sync_copy(y_hbm.at[pl.ds(s * STRIPE, STRIPE)],
                        ytab_sh.at[pl.ds(s * STRIPE, STRIPE)])
        plsc.subcore_barrier()

        gt = gbuf if bf16_gather else rows_v

        def g_src(p):
            return ytab_sh

        def g_start(j, p):
            pltpu.async_copy(g_src(p).at[src_v.at[j]], gt.at[p], sems[p])

        def g_wait(j, p):
            pltpu.make_async_copy(g_src(p).at[src_v.at[j]], gt.at[p], sems[p]).wait()

        def s_start(j, p):
            pltpu.async_copy(rows_v.at[p], acc_sh.at[dst_v.at[j]], sems[p], add=True)

        def s_wait(j, p):
            pltpu.make_async_copy(rows_v.at[p], acc_sh.at[dst_v.at[j]], sems[p]).wait()

        mask_hi = jnp.full((16,), -65536, jnp.int32)

        def convert(p):
            def cbody(i, _):
                for g in range(D // 32):
                    w = gbuf[p, i, pl.ds(16 * g, 16)]
                    lo = lax.bitcast_convert_type(w << 16, jnp.float32)
                    hi = lax.bitcast_convert_type(w & mask_hi, jnp.float32)
                    rows_v[p, i, pl.ds(32 * g, 16)] = lo
                    rows_v[p, i, pl.ds(32 * g + 16, 16)] = hi
                return 0

            lax.fori_loop(0, CHUNK, cbody, 0)

        def run(base, n):
            pltpu.sync_copy(src_hbm.at[pl.ds(base, n)], src_v.at[pl.ds(0, n)])
            pltpu.sync_copy(dst_hbm.at[pl.ds(base, n)], dst_v.at[pl.ds(0, n)])

            for p in range(NBUF):
                g_start(p, p)

            def body(g, _):
                j0 = g * NBUF
                for p in range(NBUF):
                    g_wait(j0 + p, p)
                    if bf16_gather:
                        convert(p)
                    s_start(j0 + p, p)
                for p in range(NBUF):
                    s_wait(j0 + p, p)
                    g_start(j0 + NBUF + p, p)
                return 0

            lax.fori_loop(0, n // NBUF - 1, body, 0)

            jlast = n - NBUF
            for p in range(NBUF):
                g_wait(jlast + p, p)
                if bf16_gather:
                    convert(p)
                s_start(jlast + p, p)
            for p in range(NBUF):
                s_wait(jlast + p, p)

        if CPW0 > 0:
            @pl.when(c == 0)
            def _():
                run(s * CPW0, CPW0)

        if CPW1 > 0:
            @pl.when(c == 1)
            def _():
                run(16 * CPW0 + s * CPW1, CPW1)

        plsc.subcore_barrier()
        pltpu.sync_copy(acc_sh.at[pl.ds(s * STRIPE, STRIPE)],
                        out_hbm.at[c, pl.ds(s * STRIPE, STRIPE)])

    return edge_kernel


_deg_scatter = _make_deg_kernel()
_edge_scatter64 = _make_edge_scatter(64, bf16_gather=True)
_edge_scatter16 = _make_edge_scatter(16, bf16_gather=False)


def _tc0_body(x_ref, w1_ref, h_ref):
    h_ref[...] = jnp.dot(x_ref[...], w1_ref[...], preferred_element_type=jnp.float32)


def _tc1_body(degp_ref, h_ref, dis_ref, y1_ref, y1bf_ref):
    deg = degp_ref[0, :, 0:1] + degp_ref[1, :, 0:1] + 1.0
    dis = lax.rsqrt(deg)
    dis_ref[...] = jnp.broadcast_to(dis, (NPAD, 16))
    y1 = dis * h_ref[...]
    y1_ref[...] = y1
    b16 = lax.bitcast_convert_type(y1.astype(jnp.bfloat16), jnp.int16)
    w32 = b16.astype(jnp.int32) & 0xFFFF
    words = [w32[:, 32 * g:32 * g + 16] | (w32[:, 32 * g + 16:32 * g + 32] << 16)
             for g in range(2)]
    y1bf_ref[...] = jnp.concatenate(words, axis=1)


def _tc2_body(m1p_ref, y1_ref, dis_ref, b1_ref, w2_ref, y2_ref):
    dis = dis_ref[:, 0:1]
    pre = dis * (m1p_ref[0] + m1p_ref[1] + y1_ref[...]) + b1_ref[...]
    h1 = jnp.maximum(pre, 0.0)
    g = jnp.dot(h1, w2_ref[...], preferred_element_type=jnp.float32)
    y2_ref[...] = dis * g


def _tc3_body(m2p_ref, y2_ref, dis_ref, b2_ref, z_ref):
    dis = dis_ref[:, 0:1]
    z_ref[...] = dis * (m2p_ref[0] + m2p_ref[1] + y2_ref[...]) + b2_ref[...]


@jax.jit
def kernel(x, edge_index, W1, b1, W2, b2):
    f32 = jnp.float32
    xp = jnp.zeros((NPAD, 128), f32).at[:N].set(x)
    w1p = jnp.zeros((128, 64), f32).at[:, :50].set(W1)
    b1p = jnp.zeros((1, 64), f32).at[0, :50].set(b1)
    w2p = jnp.zeros((64, 16), f32).at[:50, :10].set(W2)
    b2p = jnp.zeros((1, 16), f32).at[0, :10].set(b2)

    pad = jnp.full((EPAD - E,), DUMMY, jnp.int32)
    src2d = jnp.concatenate([edge_index[0].astype(jnp.int32), pad]).reshape(EPAD // CHUNK, CHUNK)
    dst2d = jnp.concatenate([edge_index[1].astype(jnp.int32), pad]).reshape(EPAD // CHUNK, CHUNK)

    degp = _deg_scatter(dst2d)

    h = pl.pallas_call(
        _tc0_body,
        out_shape=jax.ShapeDtypeStruct((NPAD, 64), f32),
    )(xp, w1p)

    dis, y1, y1bf = pl.pallas_call(
        _tc1_body,
        out_shape=(jax.ShapeDtypeStruct((NPAD, 16), f32),
                   jax.ShapeDtypeStruct((NPAD, 64), f32),
                   jax.ShapeDtypeStruct((NPAD, 32), jnp.int32)),
    )(degp, h)

    m1p = _edge_scatter64(y1bf, src2d, dst2d)

    y2 = pl.pallas_call(
        _tc2_body,
        out_shape=jax.ShapeDtypeStruct((NPAD, 16), f32),
    )(m1p, y1, dis, b1p, w2p)

    m2p = _edge_scatter16(y2, src2d, dst2d)

    z = pl.pallas_call(
        _tc3_body,
        out_shape=jax.ShapeDtypeStruct((NPAD, 16), f32),
    )(m2p, y2, dis, b2p)

    return z[:N, :10]


# speedup vs baseline: 1.0635x; 1.0009x over previous
import functools

import jax
import jax.numpy as jnp
from jax import lax
from jax.experimental import pallas as pl
from jax.experimental.pallas import tpu as pltpu
from jax.experimental.pallas import tpu_sc as plsc

N = 10000
NPAD = 10240
E = 320000
CHUNK = 128
NW = 32
CPW = 80
EPAD = NW * CPW * CHUNK
DUMMY = N
STRIPE = NPAD // 16

_MESH = plsc.VectorSubcoreMesh(core_axis_name="c", subcore_axis_name="s")


def _fill_rows(ref, n_rows, n_cols, value):
    vec = jnp.full((16,), value, jnp.float32)

    def body(i, _):
        for l in range(n_cols // 16):
            ref[i, pl.ds(16 * l, 16)] = vec
        return 0

    lax.fori_loop(0, n_rows, body, 0)


def _make_deg_kernel():
    @functools.partial(
        pl.kernel,
        out_type=jax.ShapeDtypeStruct((2, NPAD, 16), jnp.float32),
        mesh=_MESH,
        compiler_params=pltpu.CompilerParams(use_tc_tiling_on_sc=False),
        scratch_types=[
            pltpu.VMEM((CPW, CHUNK), jnp.int32),
            pltpu.VMEM((CHUNK, 16), jnp.float32),
            pltpu.VMEM((CHUNK, 16), jnp.float32),
            pltpu.VMEM_SHARED((NPAD, 16), jnp.float32),
            pltpu.SemaphoreType.DMA,
        ],
    )
    def deg_kernel(dst_hbm, out_hbm, dst_v, ones_v, zero_v, acc_sh, dsem):
        c = lax.axis_index("c")
        s = lax.axis_index("s")
        wid = c * 16 + s

        _fill_rows(zero_v, CHUNK, 16, 0.0)
        _fill_rows(ones_v, CHUNK, 16, 1.0)
        for t in range(STRIPE // CHUNK):
            pltpu.sync_copy(zero_v, acc_sh.at[pl.ds(s * STRIPE + t * CHUNK, CHUNK)])
        plsc.subcore_barrier()

        pltpu.sync_copy(dst_hbm.at[pl.ds(wid * CPW, CPW)], dst_v)

        def body(j, _):
            pltpu.async_copy(ones_v, acc_sh.at[dst_v.at[j]], dsem, add=True)
            return 0

        lax.fori_loop(0, CPW, body, 0)

        def drain(j, _):
            pltpu.make_async_copy(ones_v, acc_sh.at[dst_v.at[j]], dsem).wait()
            return 0

        lax.fori_loop(0, CPW, drain, 0)
        plsc.subcore_barrier()
        pltpu.sync_copy(acc_sh.at[pl.ds(s * STRIPE, STRIPE)],
                        out_hbm.at[c, pl.ds(s * STRIPE, STRIPE)])

    return deg_kernel


NBUF = 4
CPW0 = 80
CPW1 = 80
CPWMAX = max(CPW0, CPW1)


def _make_edge_scatter(D, bf16_gather):
    gdt = jnp.int32 if bf16_gather else jnp.float32
    @functools.partial(
        pl.kernel,
        out_type=jax.ShapeDtypeStruct((2, NPAD, D), jnp.float32),
        mesh=_MESH,
        compiler_params=pltpu.CompilerParams(use_tc_tiling_on_sc=False),
        scratch_types=[
            pltpu.VMEM((CPWMAX, CHUNK), jnp.int32),
            pltpu.VMEM((CPWMAX, CHUNK), jnp.int32),
            pltpu.VMEM((NBUF, CHUNK, D), jnp.float32),
            pltpu.VMEM((NBUF, CHUNK, D // 2) if bf16_gather else (1, 16, D), gdt),
            pltpu.VMEM_SHARED((NPAD, D), jnp.float32),
            pltpu.VMEM_SHARED((NPAD, D // 2 if bf16_gather else D), gdt),
        ] + [pltpu.SemaphoreType.DMA] * NBUF,
    )
    def edge_kernel(y_hbm, src_hbm, dst_hbm, out_hbm, src_v, dst_v, rows_v, gbuf, acc_sh, ytab_sh, *sems):
        c = lax.axis_index("c")
        s = lax.axis_index("s")

        vec = jnp.zeros((16,), jnp.float32)

        def zbody(i, _):
            for l in range(D // 16):
                rows_v[0, i, pl.ds(16 * l, 16)] = vec
            return 0

        lax.fori_loop(0, CHUNK, zbody, 0)
        for t in range(STRIPE // CHUNK):
            pltpu.sync_copy(rows_v.at[0], acc_sh.at[pl.ds(s * STRIPE + t * CHUNK, CHUNK)])
        pltpu.sync_copy(y_hbm.at[pl.ds(s * STRIPE, STRIPE)],
                        ytab_sh.at[pl.ds(s * STRIPE, STRIPE)])
        plsc.subcore_barrier()

        gt = gbuf if bf16_gather else rows_v

        def g_src(p):
            return ytab_sh

        def g_start(j, p):
            pltpu.async_copy(g_src(p).at[src_v.at[j]], gt.at[p], sems[p])

        def g_wait(j, p):
            pltpu.make_async_copy(g_src(p).at[src_v.at[j]], gt.at[p], sems[p]).wait()

        def s_start(j, p):
            pltpu.async_copy(rows_v.at[p], acc_sh.at[dst_v.at[j]], sems[p], add=True)

        def s_wait(j, p):
            pltpu.make_async_copy(rows_v.at[p], acc_sh.at[dst_v.at[j]], sems[p]).wait()

        mask_hi = jnp.full((16,), -65536, jnp.int32)

        def convert(p):
            def cbody(i, _):
                for g in range(D // 32):
                    w = gbuf[p, i, pl.ds(16 * g, 16)]
                    lo = lax.bitcast_convert_type(w << 16, jnp.float32)
                    hi = lax.bitcast_convert_type(w & mask_hi, jnp.float32)
                    rows_v[p, i, pl.ds(32 * g, 16)] = lo
                    rows_v[p, i, pl.ds(32 * g + 16, 16)] = hi
                return 0

            lax.fori_loop(0, CHUNK, cbody, 0)

        def run(base, n):
            pltpu.sync_copy(src_hbm.at[pl.ds(base, n)], src_v.at[pl.ds(0, n)])
            pltpu.sync_copy(dst_hbm.at[pl.ds(base, n)], dst_v.at[pl.ds(0, n)])

            for p in range(NBUF):
                g_start(p, p)

            def body(g, _):
                j0 = g * NBUF
                for p in range(NBUF):
                    g_wait(j0 + p, p)
                    if bf16_gather:
                        convert(p)
                    s_start(j0 + p, p)
                for p in range(NBUF):
                    s_wait(j0 + p, p)
                    g_start(j0 + NBUF + p, p)
                return 0

            lax.fori_loop(0, n // NBUF - 1, body, 0)

            jlast = n - NBUF
            for p in range(NBUF):
                g_wait(jlast + p, p)
                if bf16_gather:
                    convert(p)
                s_start(jlast + p, p)
            for p in range(NBUF):
                s_wait(jlast + p, p)

        if CPW0 > 0:
            @pl.when(c == 0)
            def _():
                run(s * CPW0, CPW0)

        if CPW1 > 0:
            @pl.when(c == 1)
            def _():
                run(16 * CPW0 + s * CPW1, CPW1)

        plsc.subcore_barrier()
        pltpu.sync_copy(acc_sh.at[pl.ds(s * STRIPE, STRIPE)],
                        out_hbm.at[c, pl.ds(s * STRIPE, STRIPE)])

    return edge_kernel


_deg_scatter = _make_deg_kernel()
_edge_scatter64 = _make_edge_scatter(64, bf16_gather=True)
_edge_scatter16 = _make_edge_scatter(16, bf16_gather=False)


def _tc0_body(x_ref, w1_ref, h_ref):
    h_ref[pl.ds(0, N), :] = jnp.dot(x_ref[...], w1_ref[...],
                                    preferred_element_type=jnp.float32)
    h_ref[pl.ds(N, NPAD - N), :] = jnp.zeros((NPAD - N, 64), jnp.float32)


def _tc1_body(degp_ref, h_ref, dis_ref, y1_ref, y1bf_ref):
    deg = degp_ref[0, :, 0:1] + degp_ref[1, :, 0:1] + 1.0
    dis = lax.rsqrt(deg)
    dis_ref[...] = jnp.broadcast_to(dis, (NPAD, 16))
    y1 = dis * h_ref[...]
    y1_ref[...] = y1
    b16 = lax.bitcast_convert_type(y1.astype(jnp.bfloat16), jnp.int16)
    w32 = b16.astype(jnp.int32) & 0xFFFF
    words = [w32[:, 32 * g:32 * g + 16] | (w32[:, 32 * g + 16:32 * g + 32] << 16)
             for g in range(2)]
    y1bf_ref[...] = jnp.concatenate(words, axis=1)


def _tc2_body(m1p_ref, y1_ref, dis_ref, b1_ref, w2_ref, y2_ref):
    dis = dis_ref[:, 0:1]
    pre = dis * (m1p_ref[0] + m1p_ref[1] + y1_ref[...]) + b1_ref[...]
    h1 = jnp.maximum(pre, 0.0)
    g = jnp.dot(h1, w2_ref[...], preferred_element_type=jnp.float32)
    y2_ref[...] = dis * g


def _tc3_body(m2p_ref, y2_ref, dis_ref, b2_ref, z_ref):
    dis = dis_ref[:, 0:1]
    z_ref[...] = dis * (m2p_ref[0] + m2p_ref[1] + y2_ref[...]) + b2_ref[...]


@jax.jit
def kernel(x, edge_index, W1, b1, W2, b2):
    f32 = jnp.float32
    w1p = jnp.zeros((128, 64), f32).at[:, :50].set(W1)
    b1p = jnp.zeros((1, 64), f32).at[0, :50].set(b1)
    w2p = jnp.zeros((64, 16), f32).at[:50, :10].set(W2)
    b2p = jnp.zeros((1, 16), f32).at[0, :10].set(b2)

    pad = jnp.full((EPAD - E,), DUMMY, jnp.int32)
    src2d = jnp.concatenate([edge_index[0].astype(jnp.int32), pad]).reshape(EPAD // CHUNK, CHUNK)
    dst2d = jnp.concatenate([edge_index[1].astype(jnp.int32), pad]).reshape(EPAD // CHUNK, CHUNK)

    degp = _deg_scatter(dst2d)

    h = pl.pallas_call(
        _tc0_body,
        out_shape=jax.ShapeDtypeStruct((NPAD, 64), f32),
    )(x, w1p)

    dis, y1, y1bf = pl.pallas_call(
        _tc1_body,
        out_shape=(jax.ShapeDtypeStruct((NPAD, 16), f32),
                   jax.ShapeDtypeStruct((NPAD, 64), f32),
                   jax.ShapeDtypeStruct((NPAD, 32), jnp.int32)),
    )(degp, h)

    m1p = _edge_scatter64(y1bf, src2d, dst2d)

    y2 = pl.pallas_call(
        _tc2_body,
        out_shape=jax.ShapeDtypeStruct((NPAD, 16), f32),
    )(m1p, y1, dis, b1p, w2p)

    m2p = _edge_scatter16(y2, src2d, dst2d)

    z = pl.pallas_call(
        _tc3_body,
        out_shape=jax.ShapeDtypeStruct((NPAD, 16), f32),
    )(m2p, y2, dis, b2p)

    return z[:N, :10]
